# Initial kernel scaffold; baseline (speedup 1.0000x reference)
import jax
import jax.numpy as jnp
from jax.experimental import pallas as pl


def kernel(xyz, features, params):
    raise NotImplementedError("write your pallas kernel here")


# speedup vs baseline: 1.2588x; 1.2588x over previous
import functools

import jax
import jax.numpy as jnp
from jax.experimental import pallas as pl

_ENC = [32, 64, 128, 256, 512]
_NPOINTS = (2048, 512, 128, 32)
_RADII = (0.1, 0.2, 0.4, 0.8)
_NSAMPLE = 32
_EPS = 1e-5


def _fold_bn(w, p):
    scale = p["gamma"] / jnp.sqrt(p["var"] + _EPS)
    return w * scale[None, :], (p["beta"] - p["mean"] * scale)[None, :]


def _fps(xyz, npoint):
    n = xyz.shape[0]

    def step(carry, _):
        dists, last_idx = carry
        last = xyz[last_idx]
        d = jnp.sum((xyz - last) ** 2, axis=-1)
        dists = jnp.minimum(dists, d)
        nxt = jnp.argmax(dists).astype(jnp.int32)
        return (dists, nxt), nxt

    init = (jnp.full((n,), 1e10, jnp.float32), jnp.int32(0))
    _, idxs = jax.lax.scan(step, init, None, length=npoint - 1)
    return jnp.concatenate([jnp.zeros((1,), jnp.int32), idxs])


def _ball_query(centers, xyz, radius):
    n = xyz.shape[0]
    d2 = (
        jnp.sum(centers**2, axis=1, keepdims=True)
        - 2.0 * centers @ xyz.T
        + jnp.sum(xyz**2, axis=1)[None, :]
    )
    d2e = jnp.sum((centers[:, None, :] - xyz[None, :, :]) ** 2, axis=-1)
    del d2
    keys = jnp.where(d2e <= radius * radius, jnp.arange(n, dtype=jnp.int32)[None, :], n)
    neg_top, _ = jax.lax.top_k(-keys, _NSAMPLE)
    ksort = -neg_top
    valid = ksort < n
    return jnp.where(valid, ksort, ksort[:, :1])


def _stem_kernel(x_ref, w_ref, b_ref, o_ref):
    o_ref[...] = jnp.maximum(
        jnp.dot(x_ref[...], w_ref[...], preferred_element_type=jnp.float32)
        + b_ref[...],
        0.0,
    )


def _sa_kernel(g_ref, f_ref, w0_ref, b0_ref, w1_ref, b1_ref, ws_ref, bs_ref, o_ref):
    blk, s, c3 = g_ref.shape
    g = g_ref[...].reshape(blk * s, c3)
    h = jnp.maximum(
        jnp.dot(g, w0_ref[...], preferred_element_type=jnp.float32) + b0_ref[...],
        0.0,
    )
    h = jnp.dot(h, w1_ref[...], preferred_element_type=jnp.float32) + b1_ref[...]
    cout = h.shape[-1]
    pooled = jnp.max(h.reshape(blk, s, cout), axis=1)
    skip = (
        jnp.dot(f_ref[...], ws_ref[...], preferred_element_type=jnp.float32)
        + bs_ref[...]
    )
    o_ref[...] = jnp.maximum(pooled + skip, 0.0)


def _fp_kernel(
    xf_ref, xc_ref, fc_ref, ff_ref, w0a_ref, w0b_ref, b0_ref, w1_ref, b1_ref, o_ref
):
    xf = xf_ref[...]
    xc = xc_ref[...]
    m = xc.shape[0]
    d2 = jnp.zeros((xf.shape[0], m), jnp.float32)
    for c in range(3):
        diff = xf[:, c : c + 1] - xc[:, c][None, :]
        d2 = d2 + diff * diff
    iota = jax.lax.broadcasted_iota(jnp.int32, (1, m), 1)
    rem = d2
    w = jnp.zeros_like(d2)
    for _ in range(3):
        mk = jnp.min(rem, axis=1, keepdims=True)
        ik = jnp.argmin(rem, axis=1).astype(jnp.int32)[:, None]
        oh = (iota == ik).astype(jnp.float32)
        w = w + oh / (mk + 1e-8)
        rem = rem + oh * 1e10
    w = w / jnp.sum(w, axis=1, keepdims=True)
    interp = jnp.dot(w, fc_ref[...], preferred_element_type=jnp.float32)
    h = jnp.maximum(
        jnp.dot(interp, w0a_ref[...], preferred_element_type=jnp.float32)
        + jnp.dot(ff_ref[...], w0b_ref[...], preferred_element_type=jnp.float32)
        + b0_ref[...],
        0.0,
    )
    o_ref[...] = jnp.maximum(
        jnp.dot(h, w1_ref[...], preferred_element_type=jnp.float32) + b1_ref[...],
        0.0,
    )


def _head_kernel(x_ref, w1_ref, b1_ref, w2_ref, b2_ref, o_ref):
    h = jnp.maximum(
        jnp.dot(x_ref[...], w1_ref[...], preferred_element_type=jnp.float32)
        + b1_ref[...],
        0.0,
    )
    o_ref[...] = (
        jnp.dot(h, w2_ref[...], preferred_element_type=jnp.float32) + b2_ref[...]
    )


def _full_spec(shape):
    return pl.BlockSpec(shape, lambda i: tuple(0 for _ in shape))


def _stem_call(x, w, b):
    n, c = x.shape[0], w.shape[1]
    return pl.pallas_call(
        _stem_kernel,
        out_shape=jax.ShapeDtypeStruct((n, c), jnp.float32),
    )(x, w, b)


def _sa_call(g, ffps, w0, b0, w1, b1, ws, bs):
    npoint, s, c3 = g.shape
    cin = ffps.shape[1]
    cout = w0.shape[1]
    blk = min(npoint, 512)
    grid = (npoint // blk,)
    return pl.pallas_call(
        _sa_kernel,
        grid=grid,
        in_specs=[
            pl.BlockSpec((blk, s, c3), lambda i: (i, 0, 0)),
            pl.BlockSpec((blk, cin), lambda i: (i, 0)),
            _full_spec(w0.shape),
            _full_spec(b0.shape),
            _full_spec(w1.shape),
            _full_spec(b1.shape),
            _full_spec(ws.shape),
            _full_spec(bs.shape),
        ],
        out_specs=pl.BlockSpec((blk, cout), lambda i: (i, 0)),
        out_shape=jax.ShapeDtypeStruct((npoint, cout), jnp.float32),
    )(g, ffps, w0, b0, w1, b1, ws, bs)


def _fp_call(xyz_fine, xyz_coarse, f_fine, f_coarse, w0a, w0b, b0, w1, b1):
    nf = xyz_fine.shape[0]
    cout = w1.shape[1]
    blk = min(nf, 512)
    grid = (nf // blk,)
    cf_fine = f_fine.shape[1]
    return pl.pallas_call(
        _fp_kernel,
        grid=grid,
        in_specs=[
            pl.BlockSpec((blk, 3), lambda i: (i, 0)),
            _full_spec(xyz_coarse.shape),
            _full_spec(f_coarse.shape),
            pl.BlockSpec((blk, cf_fine), lambda i: (i, 0)),
            _full_spec(w0a.shape),
            _full_spec(w0b.shape),
            _full_spec(b0.shape),
            _full_spec(w1.shape),
            _full_spec(b1.shape),
        ],
        out_specs=pl.BlockSpec((blk, cout), lambda i: (i, 0)),
        out_shape=jax.ShapeDtypeStruct((nf, cout), jnp.float32),
    )(xyz_fine, xyz_coarse, f_coarse, f_fine, w0a, w0b, b0, w1, b1)


def _head_call(x, w1, b1, w2, b2):
    n = x.shape[0]
    return pl.pallas_call(
        _head_kernel,
        out_shape=jax.ShapeDtypeStruct((n, w2.shape[1]), jnp.float32),
    )(x, w1, b1, w2, b2)


def _single(fp, xyz, feats):
    f = _stem_call(feats, fp["stem_w"], fp["stem_b"])
    xyz_list = [xyz]
    feat_list = [f]
    cx, cf = xyz, f
    for i in range(4):
        e = fp["enc"][i]
        fps_idx = _fps(cx, _NPOINTS[i])
        new_xyz = cx[fps_idx]
        nbr = _ball_query(new_xyz, cx, _RADII[i])
        dp = (cx[nbr] - new_xyz[:, None, :]) / _RADII[i]
        g = jnp.concatenate([dp, cf[nbr]], axis=-1)
        cf = _sa_call(
            g, cf[fps_idx], e["w0"], e["b0"], e["w1"], e["b1"], e["ws"], e["bs"]
        )
        cx = new_xyz
        xyz_list.append(cx)
        feat_list.append(cf)
    f_cur = feat_list[-1]
    for i in range(4):
        fine = 3 - i
        d = fp["dec"][i]
        f_cur = _fp_call(
            xyz_list[fine],
            xyz_list[fine + 1],
            feat_list[fine],
            f_cur,
            d["w0a"],
            d["w0b"],
            d["b0"],
            d["w1"],
            d["b1"],
        )
    return _head_call(f_cur, fp["hw1"], fp["hb1"], fp["hw2"], fp["hb2"])


def kernel(xyz, features, params):
    stem_w, stem_b = _fold_bn(params["stem_w"], params["stem_bn"])
    folded = {"stem_w": stem_w, "stem_b": stem_b, "enc": [], "dec": []}
    for i in range(4):
        e = params["enc"][i]
        w0, b0 = _fold_bn(e["mlp0_w"], e["mlp0_bn"])
        w1, b1 = _fold_bn(e["mlp1_w"], e["mlp1_bn"])
        ws, bs = _fold_bn(e["skip_w"], e["skip_bn"])
        folded["enc"].append(
            {"w0": w0, "b0": b0, "w1": w1, "b1": b1, "ws": ws, "bs": bs}
        )
    for i in range(4):
        d = params["dec"][i]
        fine = 3 - i
        c_coarse = _ENC[fine + 1]
        w0, b0 = _fold_bn(d["w0"], d["bn0"])
        w1, b1 = _fold_bn(d["w1"], d["bn1"])
        folded["dec"].append(
            {
                "w0a": w0[:c_coarse],
                "w0b": w0[c_coarse:],
                "b0": b0,
                "w1": w1,
                "b1": b1,
            }
        )
    hw1, hb1 = _fold_bn(params["head_w1"], params["head_bn"])
    folded["hw1"] = hw1
    folded["hb1"] = hb1
    folded["hw2"] = params["head_w2"]
    folded["hb2"] = params["head_b2"][None, :]
    return jax.vmap(lambda a, b: _single(folded, a, b))(xyz, features)


# speedup vs baseline: 1.9383x; 1.5397x over previous
import functools

import jax
import jax.numpy as jnp
from jax.experimental import pallas as pl

_ENC = [32, 64, 128, 256, 512]
_NPOINTS = (2048, 512, 128, 32)
_RADII = (0.1, 0.2, 0.4, 0.8)
_NSAMPLE = 32
_EPS = 1e-5


def _fold_bn(w, p):
    scale = p["gamma"] / jnp.sqrt(p["var"] + _EPS)
    return w * scale[None, :], (p["beta"] - p["mean"] * scale)[None, :]


def _fps_kernel(xyz_ref, xr_ref, o_ref):
    n = xyz_ref.shape[0]
    npoint = o_ref.shape[0]
    _, rows, cols = xr_ref.shape
    x0 = xr_ref[0]
    x1 = xr_ref[1]
    x2 = xr_ref[2]
    flat = (
        jax.lax.broadcasted_iota(jnp.int32, (rows, cols), 0) * cols
        + jax.lax.broadcasted_iota(jnp.int32, (rows, cols), 1)
    )
    o_ref[0:1, 0:1] = jnp.zeros((1, 1), jnp.int32)

    def body(k, carry):
        dists, last = carry
        row = xyz_ref[pl.ds(last, 1), :]
        d = (x0 - row[0, 0]) ** 2 + (x1 - row[0, 1]) ** 2 + (x2 - row[0, 2]) ** 2
        dists = jnp.minimum(dists, d)
        glob = jnp.max(dists)
        nxt = jnp.min(jnp.where(dists == glob, flat, n)).astype(jnp.int32)
        o_ref[pl.ds(k, 1), 0:1] = jnp.reshape(nxt, (1, 1))
        return dists, nxt

    jax.lax.fori_loop(
        1, npoint, body, (jnp.full((rows, cols), 1e10, jnp.float32), jnp.int32(0))
    )


def _fps(xyz, npoint):
    n = xyz.shape[0]
    xr = xyz.T.reshape(3, 8, n // 8)
    out = pl.pallas_call(
        _fps_kernel,
        out_shape=jax.ShapeDtypeStruct((npoint, 1), jnp.int32),
    )(xyz, xr)
    return out[:, 0]


def _ball_query(centers, xyz, radius):
    n = xyz.shape[0]
    d2 = (
        jnp.sum(centers**2, axis=1, keepdims=True)
        - 2.0 * centers @ xyz.T
        + jnp.sum(xyz**2, axis=1)[None, :]
    )
    d2e = jnp.sum((centers[:, None, :] - xyz[None, :, :]) ** 2, axis=-1)
    del d2
    keys = jnp.where(d2e <= radius * radius, jnp.arange(n, dtype=jnp.int32)[None, :], n)
    neg_top, _ = jax.lax.top_k(-keys, _NSAMPLE)
    ksort = -neg_top
    valid = ksort < n
    return jnp.where(valid, ksort, ksort[:, :1])


def _stem_kernel(x_ref, w_ref, b_ref, o_ref):
    o_ref[...] = jnp.maximum(
        jnp.dot(x_ref[...], w_ref[...], preferred_element_type=jnp.float32)
        + b_ref[...],
        0.0,
    )


def _sa_kernel(g_ref, f_ref, w0_ref, b0_ref, w1_ref, b1_ref, ws_ref, bs_ref, o_ref):
    blk, s, c3 = g_ref.shape
    g = g_ref[...].reshape(blk * s, c3)
    h = jnp.maximum(
        jnp.dot(g, w0_ref[...], preferred_element_type=jnp.float32) + b0_ref[...],
        0.0,
    )
    h = jnp.dot(h, w1_ref[...], preferred_element_type=jnp.float32) + b1_ref[...]
    cout = h.shape[-1]
    pooled = jnp.max(h.reshape(blk, s, cout), axis=1)
    skip = (
        jnp.dot(f_ref[...], ws_ref[...], preferred_element_type=jnp.float32)
        + bs_ref[...]
    )
    o_ref[...] = jnp.maximum(pooled + skip, 0.0)


def _fp_kernel(
    xf_ref, xc_ref, fc_ref, ff_ref, w0a_ref, w0b_ref, b0_ref, w1_ref, b1_ref, o_ref
):
    xf = xf_ref[...]
    xc = xc_ref[...]
    m = xc.shape[0]
    d2 = jnp.zeros((xf.shape[0], m), jnp.float32)
    for c in range(3):
        diff = xf[:, c : c + 1] - xc[:, c][None, :]
        d2 = d2 + diff * diff
    iota = jax.lax.broadcasted_iota(jnp.int32, (1, m), 1)
    rem = d2
    w = jnp.zeros_like(d2)
    for _ in range(3):
        mk = jnp.min(rem, axis=1, keepdims=True)
        ik = jnp.argmin(rem, axis=1).astype(jnp.int32)[:, None]
        oh = (iota == ik).astype(jnp.float32)
        w = w + oh / (mk + 1e-8)
        rem = rem + oh * 1e10
    w = w / jnp.sum(w, axis=1, keepdims=True)
    interp = jnp.dot(w, fc_ref[...], preferred_element_type=jnp.float32)
    h = jnp.maximum(
        jnp.dot(interp, w0a_ref[...], preferred_element_type=jnp.float32)
        + jnp.dot(ff_ref[...], w0b_ref[...], preferred_element_type=jnp.float32)
        + b0_ref[...],
        0.0,
    )
    o_ref[...] = jnp.maximum(
        jnp.dot(h, w1_ref[...], preferred_element_type=jnp.float32) + b1_ref[...],
        0.0,
    )


def _head_kernel(x_ref, w1_ref, b1_ref, w2_ref, b2_ref, o_ref):
    h = jnp.maximum(
        jnp.dot(x_ref[...], w1_ref[...], preferred_element_type=jnp.float32)
        + b1_ref[...],
        0.0,
    )
    o_ref[...] = (
        jnp.dot(h, w2_ref[...], preferred_element_type=jnp.float32) + b2_ref[...]
    )


def _full_spec(shape):
    return pl.BlockSpec(shape, lambda i: tuple(0 for _ in shape))


def _stem_call(x, w, b):
    n, c = x.shape[0], w.shape[1]
    return pl.pallas_call(
        _stem_kernel,
        out_shape=jax.ShapeDtypeStruct((n, c), jnp.float32),
    )(x, w, b)


def _sa_call(g, ffps, w0, b0, w1, b1, ws, bs):
    npoint, s, c3 = g.shape
    cin = ffps.shape[1]
    cout = w0.shape[1]
    blk = min(npoint, 512)
    grid = (npoint // blk,)
    return pl.pallas_call(
        _sa_kernel,
        grid=grid,
        in_specs=[
            pl.BlockSpec((blk, s, c3), lambda i: (i, 0, 0)),
            pl.BlockSpec((blk, cin), lambda i: (i, 0)),
            _full_spec(w0.shape),
            _full_spec(b0.shape),
            _full_spec(w1.shape),
            _full_spec(b1.shape),
            _full_spec(ws.shape),
            _full_spec(bs.shape),
        ],
        out_specs=pl.BlockSpec((blk, cout), lambda i: (i, 0)),
        out_shape=jax.ShapeDtypeStruct((npoint, cout), jnp.float32),
    )(g, ffps, w0, b0, w1, b1, ws, bs)


def _fp_call(xyz_fine, xyz_coarse, f_fine, f_coarse, w0a, w0b, b0, w1, b1):
    nf = xyz_fine.shape[0]
    cout = w1.shape[1]
    blk = min(nf, 512)
    grid = (nf // blk,)
    cf_fine = f_fine.shape[1]
    return pl.pallas_call(
        _fp_kernel,
        grid=grid,
        in_specs=[
            pl.BlockSpec((blk, 3), lambda i: (i, 0)),
            _full_spec(xyz_coarse.shape),
            _full_spec(f_coarse.shape),
            pl.BlockSpec((blk, cf_fine), lambda i: (i, 0)),
            _full_spec(w0a.shape),
            _full_spec(w0b.shape),
            _full_spec(b0.shape),
            _full_spec(w1.shape),
            _full_spec(b1.shape),
        ],
        out_specs=pl.BlockSpec((blk, cout), lambda i: (i, 0)),
        out_shape=jax.ShapeDtypeStruct((nf, cout), jnp.float32),
    )(xyz_fine, xyz_coarse, f_coarse, f_fine, w0a, w0b, b0, w1, b1)


def _head_call(x, w1, b1, w2, b2):
    n = x.shape[0]
    return pl.pallas_call(
        _head_kernel,
        out_shape=jax.ShapeDtypeStruct((n, w2.shape[1]), jnp.float32),
    )(x, w1, b1, w2, b2)


def _single(fp, xyz, feats):
    f = _stem_call(feats, fp["stem_w"], fp["stem_b"])
    xyz_list = [xyz]
    feat_list = [f]
    cx, cf = xyz, f
    for i in range(4):
        e = fp["enc"][i]
        fps_idx = _fps(cx, _NPOINTS[i])
        new_xyz = cx[fps_idx]
        nbr = _ball_query(new_xyz, cx, _RADII[i])
        dp = (cx[nbr] - new_xyz[:, None, :]) / _RADII[i]
        g = jnp.concatenate([dp, cf[nbr]], axis=-1)
        cf = _sa_call(
            g, cf[fps_idx], e["w0"], e["b0"], e["w1"], e["b1"], e["ws"], e["bs"]
        )
        cx = new_xyz
        xyz_list.append(cx)
        feat_list.append(cf)
    f_cur = feat_list[-1]
    for i in range(4):
        fine = 3 - i
        d = fp["dec"][i]
        f_cur = _fp_call(
            xyz_list[fine],
            xyz_list[fine + 1],
            feat_list[fine],
            f_cur,
            d["w0a"],
            d["w0b"],
            d["b0"],
            d["w1"],
            d["b1"],
        )
    return _head_call(f_cur, fp["hw1"], fp["hb1"], fp["hw2"], fp["hb2"])


def kernel(xyz, features, params):
    stem_w, stem_b = _fold_bn(params["stem_w"], params["stem_bn"])
    folded = {"stem_w": stem_w, "stem_b": stem_b, "enc": [], "dec": []}
    for i in range(4):
        e = params["enc"][i]
        w0, b0 = _fold_bn(e["mlp0_w"], e["mlp0_bn"])
        w1, b1 = _fold_bn(e["mlp1_w"], e["mlp1_bn"])
        ws, bs = _fold_bn(e["skip_w"], e["skip_bn"])
        folded["enc"].append(
            {"w0": w0, "b0": b0, "w1": w1, "b1": b1, "ws": ws, "bs": bs}
        )
    for i in range(4):
        d = params["dec"][i]
        fine = 3 - i
        c_coarse = _ENC[fine + 1]
        w0, b0 = _fold_bn(d["w0"], d["bn0"])
        w1, b1 = _fold_bn(d["w1"], d["bn1"])
        folded["dec"].append(
            {
                "w0a": w0[:c_coarse],
                "w0b": w0[c_coarse:],
                "b0": b0,
                "w1": w1,
                "b1": b1,
            }
        )
    hw1, hb1 = _fold_bn(params["head_w1"], params["head_bn"])
    folded["hw1"] = hw1
    folded["hb1"] = hb1
    folded["hw2"] = params["head_w2"]
    folded["hb2"] = params["head_b2"][None, :]
    return jax.vmap(lambda a, b: _single(folded, a, b))(xyz, features)


# speedup vs baseline: 1.9511x; 1.0066x over previous
import functools

import jax
import jax.numpy as jnp
from jax.experimental import pallas as pl

_ENC = [32, 64, 128, 256, 512]
_NPOINTS = (2048, 512, 128, 32)
_RADII = (0.1, 0.2, 0.4, 0.8)
_NSAMPLE = 32
_EPS = 1e-5


def _fold_bn(w, p):
    scale = p["gamma"] / jnp.sqrt(p["var"] + _EPS)
    return w * scale[None, :], (p["beta"] - p["mean"] * scale)[None, :]


def _fps_kernel(xyz_ref, xr_ref, o_ref):
    n = xyz_ref.shape[0]
    npoint = o_ref.shape[0]
    _, rows, cols = xr_ref.shape
    x0 = xr_ref[0]
    x1 = xr_ref[1]
    x2 = xr_ref[2]
    flat = (
        jax.lax.broadcasted_iota(jnp.int32, (rows, cols), 0) * cols
        + jax.lax.broadcasted_iota(jnp.int32, (rows, cols), 1)
    )
    o_ref[0:1, 0:1] = jnp.zeros((1, 1), jnp.int32)

    def body(k, carry):
        dists, last = carry
        row = xyz_ref[pl.ds(last, 1), :]
        d = (x0 - row[0, 0]) ** 2 + (x1 - row[0, 1]) ** 2 + (x2 - row[0, 2]) ** 2
        dists = jnp.minimum(dists, d)
        glob = jnp.max(dists)
        nxt = jnp.min(jnp.where(dists == glob, flat, n)).astype(jnp.int32)
        o_ref[pl.ds(k, 1), 0:1] = jnp.reshape(nxt, (1, 1))
        return dists, nxt

    jax.lax.fori_loop(
        1, npoint, body, (jnp.full((rows, cols), 1e10, jnp.float32), jnp.int32(0))
    )


def _fps(xyz, npoint):
    n = xyz.shape[0]
    xr = xyz.T.reshape(3, 8, n // 8)
    out = pl.pallas_call(
        _fps_kernel,
        out_shape=jax.ShapeDtypeStruct((npoint, 1), jnp.int32),
    )(xyz, xr)
    return out[:, 0]


def _ball_query(centers, xyz, radius):
    n = xyz.shape[0]
    d2e = (
        (centers[:, 0:1] - xyz[:, 0][None, :]) ** 2
        + (centers[:, 1:2] - xyz[:, 1][None, :]) ** 2
        + (centers[:, 2:3] - xyz[:, 2][None, :]) ** 2
    )
    keys = jnp.where(d2e <= radius * radius, jnp.arange(n, dtype=jnp.int32)[None, :], n)
    neg_top, _ = jax.lax.top_k(-keys, _NSAMPLE)
    ksort = -neg_top
    valid = ksort < n
    return jnp.where(valid, ksort, ksort[:, :1])


def _stem_kernel(x_ref, w_ref, b_ref, o_ref):
    o_ref[...] = jnp.maximum(
        jnp.dot(x_ref[...], w_ref[...], preferred_element_type=jnp.float32)
        + b_ref[...],
        0.0,
    )


def _sa_kernel(g_ref, f_ref, w0_ref, b0_ref, w1_ref, b1_ref, ws_ref, bs_ref, o_ref):
    blk, s, c3 = g_ref.shape
    g = g_ref[...].reshape(blk * s, c3)
    h = jnp.maximum(
        jnp.dot(g, w0_ref[...], preferred_element_type=jnp.float32) + b0_ref[...],
        0.0,
    )
    h = jnp.dot(h, w1_ref[...], preferred_element_type=jnp.float32) + b1_ref[...]
    cout = h.shape[-1]
    pooled = jnp.max(h.reshape(blk, s, cout), axis=1)
    skip = (
        jnp.dot(f_ref[...], ws_ref[...], preferred_element_type=jnp.float32)
        + bs_ref[...]
    )
    o_ref[...] = jnp.maximum(pooled + skip, 0.0)


def _fp_kernel(
    xf_ref, xc_ref, fc_ref, ff_ref, w0a_ref, w0b_ref, b0_ref, w1_ref, b1_ref, o_ref
):
    xf = xf_ref[...]
    xc = xc_ref[...]
    m = xc.shape[0]
    d2 = jnp.zeros((xf.shape[0], m), jnp.float32)
    for c in range(3):
        diff = xf[:, c : c + 1] - xc[:, c][None, :]
        d2 = d2 + diff * diff
    iota = jax.lax.broadcasted_iota(jnp.int32, (1, m), 1)
    rem = d2
    w = jnp.zeros_like(d2)
    for _ in range(3):
        mk = jnp.min(rem, axis=1, keepdims=True)
        ik = jnp.argmin(rem, axis=1).astype(jnp.int32)[:, None]
        oh = (iota == ik).astype(jnp.float32)
        w = w + oh / (mk + 1e-8)
        rem = rem + oh * 1e10
    w = w / jnp.sum(w, axis=1, keepdims=True)
    interp = jnp.dot(w, fc_ref[...], preferred_element_type=jnp.float32)
    h = jnp.maximum(
        jnp.dot(interp, w0a_ref[...], preferred_element_type=jnp.float32)
        + jnp.dot(ff_ref[...], w0b_ref[...], preferred_element_type=jnp.float32)
        + b0_ref[...],
        0.0,
    )
    o_ref[...] = jnp.maximum(
        jnp.dot(h, w1_ref[...], preferred_element_type=jnp.float32) + b1_ref[...],
        0.0,
    )


def _head_kernel(x_ref, w1_ref, b1_ref, w2_ref, b2_ref, o_ref):
    h = jnp.maximum(
        jnp.dot(x_ref[...], w1_ref[...], preferred_element_type=jnp.float32)
        + b1_ref[...],
        0.0,
    )
    o_ref[...] = (
        jnp.dot(h, w2_ref[...], preferred_element_type=jnp.float32) + b2_ref[...]
    )


def _full_spec(shape):
    return pl.BlockSpec(shape, lambda i: tuple(0 for _ in shape))


def _stem_call(x, w, b):
    n, c = x.shape[0], w.shape[1]
    return pl.pallas_call(
        _stem_kernel,
        out_shape=jax.ShapeDtypeStruct((n, c), jnp.float32),
    )(x, w, b)


def _sa_call(g, ffps, w0, b0, w1, b1, ws, bs):
    npoint, s, c3 = g.shape
    cin = ffps.shape[1]
    cout = w0.shape[1]
    blk = min(npoint, 512)
    grid = (npoint // blk,)
    return pl.pallas_call(
        _sa_kernel,
        grid=grid,
        in_specs=[
            pl.BlockSpec((blk, s, c3), lambda i: (i, 0, 0)),
            pl.BlockSpec((blk, cin), lambda i: (i, 0)),
            _full_spec(w0.shape),
            _full_spec(b0.shape),
            _full_spec(w1.shape),
            _full_spec(b1.shape),
            _full_spec(ws.shape),
            _full_spec(bs.shape),
        ],
        out_specs=pl.BlockSpec((blk, cout), lambda i: (i, 0)),
        out_shape=jax.ShapeDtypeStruct((npoint, cout), jnp.float32),
    )(g, ffps, w0, b0, w1, b1, ws, bs)


def _fp_call(xyz_fine, xyz_coarse, f_fine, f_coarse, w0a, w0b, b0, w1, b1):
    nf = xyz_fine.shape[0]
    cout = w1.shape[1]
    blk = min(nf, 512)
    grid = (nf // blk,)
    cf_fine = f_fine.shape[1]
    return pl.pallas_call(
        _fp_kernel,
        grid=grid,
        in_specs=[
            pl.BlockSpec((blk, 3), lambda i: (i, 0)),
            _full_spec(xyz_coarse.shape),
            _full_spec(f_coarse.shape),
            pl.BlockSpec((blk, cf_fine), lambda i: (i, 0)),
            _full_spec(w0a.shape),
            _full_spec(w0b.shape),
            _full_spec(b0.shape),
            _full_spec(w1.shape),
            _full_spec(b1.shape),
        ],
        out_specs=pl.BlockSpec((blk, cout), lambda i: (i, 0)),
        out_shape=jax.ShapeDtypeStruct((nf, cout), jnp.float32),
    )(xyz_fine, xyz_coarse, f_coarse, f_fine, w0a, w0b, b0, w1, b1)


def _head_call(x, w1, b1, w2, b2):
    n = x.shape[0]
    return pl.pallas_call(
        _head_kernel,
        out_shape=jax.ShapeDtypeStruct((n, w2.shape[1]), jnp.float32),
    )(x, w1, b1, w2, b2)


def _single(fp, xyz, feats):
    f = _stem_call(feats, fp["stem_w"], fp["stem_b"])
    xyz_list = [xyz]
    feat_list = [f]
    cx, cf = xyz, f
    for i in range(4):
        e = fp["enc"][i]
        fps_idx = _fps(cx, _NPOINTS[i])
        new_xyz = cx[fps_idx]
        nbr = _ball_query(new_xyz, cx, _RADII[i])
        dp = (cx[nbr] - new_xyz[:, None, :]) / _RADII[i]
        g = jnp.concatenate([dp, cf[nbr]], axis=-1)
        cf = _sa_call(
            g, cf[fps_idx], e["w0"], e["b0"], e["w1"], e["b1"], e["ws"], e["bs"]
        )
        cx = new_xyz
        xyz_list.append(cx)
        feat_list.append(cf)
    f_cur = feat_list[-1]
    for i in range(4):
        fine = 3 - i
        d = fp["dec"][i]
        f_cur = _fp_call(
            xyz_list[fine],
            xyz_list[fine + 1],
            feat_list[fine],
            f_cur,
            d["w0a"],
            d["w0b"],
            d["b0"],
            d["w1"],
            d["b1"],
        )
    return _head_call(f_cur, fp["hw1"], fp["hb1"], fp["hw2"], fp["hb2"])


def kernel(xyz, features, params):
    stem_w, stem_b = _fold_bn(params["stem_w"], params["stem_bn"])
    folded = {"stem_w": stem_w, "stem_b": stem_b, "enc": [], "dec": []}
    for i in range(4):
        e = params["enc"][i]
        w0, b0 = _fold_bn(e["mlp0_w"], e["mlp0_bn"])
        w1, b1 = _fold_bn(e["mlp1_w"], e["mlp1_bn"])
        ws, bs = _fold_bn(e["skip_w"], e["skip_bn"])
        folded["enc"].append(
            {"w0": w0, "b0": b0, "w1": w1, "b1": b1, "ws": ws, "bs": bs}
        )
    for i in range(4):
        d = params["dec"][i]
        fine = 3 - i
        c_coarse = _ENC[fine + 1]
        w0, b0 = _fold_bn(d["w0"], d["bn0"])
        w1, b1 = _fold_bn(d["w1"], d["bn1"])
        folded["dec"].append(
            {
                "w0a": w0[:c_coarse],
                "w0b": w0[c_coarse:],
                "b0": b0,
                "w1": w1,
                "b1": b1,
            }
        )
    hw1, hb1 = _fold_bn(params["head_w1"], params["head_bn"])
    folded["hw1"] = hw1
    folded["hb1"] = hb1
    folded["hw2"] = params["head_w2"]
    folded["hb2"] = params["head_b2"][None, :]
    return jax.vmap(lambda a, b: _single(folded, a, b))(xyz, features)
